# Initial kernel scaffold; baseline (speedup 1.0000x reference)
import functools

import jax
import jax.numpy as jnp
from jax import lax
from jax.experimental import pallas as pl
from jax.experimental.pallas import tpu as pltpu
from jax.experimental.pallas import tpu_sc as plsc

B = 16384
D = 128
K = 20
R = K + 1
NC = 2
NS = 16
NW = NC * NS
BPW = B // NW
S = 16
NSUB = BPW // S
ROWS = S * R
IDXW = 112
NIDX = ROWS // IDXW


def _sc_scores(tid, cid, ttab, ctab):
    mesh = plsc.VectorSubcoreMesh(core_axis_name="c", subcore_axis_name="s")

    @functools.partial(
        pl.kernel,
        out_type=(
            jax.ShapeDtypeStruct((B,), jnp.float32),
            jax.ShapeDtypeStruct((B * K,), jnp.float32),
        ),
        mesh=mesh,
        scratch_types=[
            pltpu.VMEM((BPW,), jnp.int32),
            pltpu.VMEM((NSUB, NIDX, IDXW), jnp.int32),
            pltpu.VMEM((S, D), jnp.float32),
            pltpu.VMEM((ROWS, D), jnp.float32),
            pltpu.VMEM((S,), jnp.float32),
            pltpu.VMEM((S * K,), jnp.float32),
            pltpu.SemaphoreType.DMA,
        ],
    )
    def k(tid_h, cid_h, ttab_h, ctab_h, pos_h, neg_h,
          tidx, cidx, trows, crows, posv, negv, sem):
        wid = lax.axis_index("s") * NC + lax.axis_index("c")
        pltpu.sync_copy(tid_h.at[wid], tidx)
        pltpu.sync_copy(cid_h.at[wid], cidx)

        def sub(si, carry):
            b0 = wid * BPW + si * S
            cp_t = pltpu.async_copy(
                ttab_h.at[tidx.at[pl.ds(si * S, S)]], trows, sem)
            cps = [
                pltpu.async_copy(
                    ctab_h.at[cidx.at[si, i]],
                    crows.at[pl.ds(i * IDXW, IDXW)], sem)
                for i in range(NIDX)
            ]
            cp_t.wait()
            for cp in cps:
                cp.wait()

            def per_b(b, c2):
                t = [trows[b, pl.ds(j * 16, 16)] for j in range(8)]
                for r in range(R):
                    row = b * R + r
                    p = [t[j] * crows[row, pl.ds(j * 16, 16)]
                         for j in range(8)]
                    q = [p[2 * j] + p[2 * j + 1] for j in range(4)]
                    acc = (q[0] + q[1]) + (q[2] + q[3])
                    sc = jnp.sum(acc)
                    if r == 0:
                        posv[b] = sc
                    else:
                        negv[b * K + (r - 1)] = sc
                return c2

            lax.fori_loop(0, S, per_b, 0)
            pltpu.sync_copy(posv, pos_h.at[pl.ds(b0, S)])
            pltpu.sync_copy(negv, neg_h.at[pl.ds(b0 * K, S * K)])
            return carry

        lax.fori_loop(0, NSUB, sub, 0)

    return k(tid, cid, ttab, ctab)


def _tc_loss(pos, neg):
    pos2 = pos.reshape(B // D, D)
    neg2 = neg.reshape(B * K // D, D)

    def body(p_ref, n_ref, o_ref):
        def logsig(x):
            return jnp.minimum(x, 0.0) - jnp.log1p(jnp.exp(-jnp.abs(x)))

        tot = jnp.sum(logsig(p_ref[...])) + jnp.sum(logsig(-n_ref[...]))
        o_ref[0, 0] = -tot / B

    out = pl.pallas_call(
        body,
        out_shape=jax.ShapeDtypeStruct((1, 1), jnp.float32),
        out_specs=pl.BlockSpec(memory_space=pltpu.SMEM),
    )(pos2, neg2)
    return out[0, 0]


def kernel(target_ids, context_ids, neg_ids, target_table, context_table):
    tid = target_ids.astype(jnp.int32).reshape(NW, BPW)
    cid = jnp.concatenate(
        [context_ids[:, None], neg_ids], axis=1).astype(jnp.int32)
    cid = cid.reshape(NW, NSUB, NIDX, IDXW)
    pos, negs = _sc_scores(tid, cid, target_table, context_table)
    return _tc_loss(pos, negs)


# speedup vs baseline: 3.2826x; 3.2826x over previous
import functools

import jax
import jax.numpy as jnp
from jax import lax
from jax.experimental import pallas as pl
from jax.experimental.pallas import tpu as pltpu
from jax.experimental.pallas import tpu_sc as plsc

B = 16384
D = 128
K = 20
R = K + 1
L = 16
NC = 2
NS = 16
NW = NC * NS
BPW = B // NW
S = 16
NSUB = BPW // S
ROWS = S * R
IDXW = 112
NIDX = ROWS // IDXW


def _sc_scores(tid, cid, ttab, ctab):
    mesh = plsc.VectorSubcoreMesh(core_axis_name="c", subcore_axis_name="s")

    @functools.partial(
        pl.kernel,
        out_type=(
            jax.ShapeDtypeStruct((B, L), jnp.float32),
            jax.ShapeDtypeStruct((B * K, L), jnp.float32),
        ),
        mesh=mesh,
        scratch_types=[
            pltpu.VMEM((BPW,), jnp.int32),
            pltpu.VMEM((NSUB, NIDX, IDXW), jnp.int32),
            pltpu.VMEM((S, D), jnp.float32),
            pltpu.VMEM((ROWS, D), jnp.float32),
            pltpu.VMEM((S, L), jnp.float32),
            pltpu.VMEM((S * K, L), jnp.float32),
            pltpu.SemaphoreType.DMA,
        ],
    )
    def k(tid_h, cid_h, ttab_h, ctab_h, pos_h, neg_h,
          tidx, cidx, trows, crows, posv, negv, sem):
        wid = lax.axis_index("s") * NC + lax.axis_index("c")
        pltpu.sync_copy(tid_h.at[wid], tidx)
        pltpu.sync_copy(cid_h.at[wid], cidx)

        def sub(si, carry):
            b0 = wid * BPW + si * S
            cp_t = pltpu.async_copy(
                ttab_h.at[tidx.at[pl.ds(si * S, S)]], trows, sem)
            cps = [
                pltpu.async_copy(
                    ctab_h.at[cidx.at[si, i]],
                    crows.at[pl.ds(i * IDXW, IDXW)], sem)
                for i in range(NIDX)
            ]
            cp_t.wait()
            for cp in cps:
                cp.wait()

            def per_b(b, c2):
                t = [trows[b, pl.ds(j * L, L)] for j in range(8)]
                for r in range(R):
                    row = b * R + r
                    p = [t[j] * crows[row, pl.ds(j * L, L)]
                         for j in range(8)]
                    q = [p[2 * j] + p[2 * j + 1] for j in range(4)]
                    acc = (q[0] + q[1]) + (q[2] + q[3])
                    if r == 0:
                        posv[b, :] = acc
                    else:
                        negv[b * K + (r - 1), :] = acc
                return c2

            lax.fori_loop(0, S, per_b, 0)
            pltpu.sync_copy(posv, pos_h.at[pl.ds(b0, S)])
            pltpu.sync_copy(negv, neg_h.at[pl.ds(b0 * K, S * K)])
            return carry

        lax.fori_loop(0, NSUB, sub, 0)

    return k(tid, cid, ttab, ctab)


GRID = 10


def _tc_loss(pos, neg):
    posr = pos.reshape(B // 8, D)
    negr = neg.reshape(B * K // 8, D)
    nblk = (B * K // 8) // GRID

    def body(p_ref, n_ref, o_ref):
        i = pl.program_id(0)
        lane = lax.broadcasted_iota(jnp.int32, (D, 8), 0)
        col = lax.broadcasted_iota(jnp.int32, (D, 8), 1)
        m = (lane // L == col).astype(jnp.float32)

        def logsig(x):
            return jnp.minimum(x, 0.0) - jnp.log1p(jnp.exp(-jnp.abs(x)))

        @pl.when(i == 0)
        def _init():
            ps = jnp.dot(p_ref[...], m, preferred_element_type=jnp.float32)
            o_ref[0, 0] = jnp.sum(logsig(ps))

        ns = jnp.dot(n_ref[...], m, preferred_element_type=jnp.float32)
        o_ref[0, 0] += jnp.sum(logsig(-ns))

        @pl.when(i == GRID - 1)
        def _fin():
            o_ref[0, 0] = -o_ref[0, 0] / B

    out = pl.pallas_call(
        body,
        grid=(GRID,),
        in_specs=[
            pl.BlockSpec((B // 8, D), lambda i: (0, 0)),
            pl.BlockSpec((nblk, D), lambda i: (i, 0)),
        ],
        out_specs=pl.BlockSpec((1, 1), lambda i: (0, 0),
                               memory_space=pltpu.SMEM),
        out_shape=jax.ShapeDtypeStruct((1, 1), jnp.float32),
    )(posr, negr)
    return out[0, 0]


def kernel(target_ids, context_ids, neg_ids, target_table, context_table):
    tid = target_ids.astype(jnp.int32).reshape(NW, BPW)
    cid = jnp.concatenate(
        [context_ids[:, None], neg_ids], axis=1).astype(jnp.int32)
    cid = cid.reshape(NW, NSUB, NIDX, IDXW)
    pos, negs = _sc_scores(tid, cid, target_table, context_table)
    return _tc_loss(pos, negs)


# speedup vs baseline: 10.9653x; 3.3404x over previous
import functools

import jax
import jax.numpy as jnp
from jax import lax
from jax.experimental import pallas as pl
from jax.experimental.pallas import tpu as pltpu
from jax.experimental.pallas import tpu_sc as plsc

B = 16384
D = 128
K = 20
R = K + 1
L = 16
NC = 2
NS = 16
NW = NC * NS
BPW = B // NW
S = 16
NSUB = BPW // S
ROWS = S * R
IDXW = 112
NIDX = ROWS // IDXW


def _sc_scores(tid, cid, ttab, ctab):
    mesh = plsc.VectorSubcoreMesh(core_axis_name="c", subcore_axis_name="s")

    @functools.partial(
        pl.kernel,
        out_type=jax.ShapeDtypeStruct((B * R,), jnp.float32),
        mesh=mesh,
        scratch_types=[
            pltpu.VMEM((BPW,), jnp.int32),
            pltpu.VMEM((NSUB, NIDX, IDXW), jnp.int32),
            pltpu.VMEM((2, S, D), jnp.float32),
            pltpu.VMEM((2, ROWS, D), jnp.float32),
            pltpu.VMEM((ROWS + L,), jnp.float32),
            pltpu.VMEM((ROWS + L,), jnp.float32),
            pltpu.SemaphoreType.DMA,
            pltpu.SemaphoreType.DMA,
            pltpu.SemaphoreType.DMA,
            pltpu.SemaphoreType.DMA,
        ],
        compiler_params=pltpu.CompilerParams(needs_layout_passes=False),
    )
    def k(tid_h, cid_h, ttab_h, ctab_h, out_h,
          tidx, cidx, trows, crows, sv0, sv1, g0, g1, o0, o1):
        wid = lax.axis_index("s") * NC + lax.axis_index("c")
        pltpu.sync_copy(tid_h.at[wid], tidx)
        pltpu.sync_copy(cid_h.at[wid], cidx)
        svs = (sv0, sv1)
        gsem = (g0, g1)
        osem = (o0, o1)
        lastlane = lax.iota(jnp.int32, L) == (L - 1)

        def mk_gather(si, p):
            cps = [pltpu.make_async_copy(
                ttab_h.at[tidx.at[pl.ds(si * S, S)]], trows.at[p], gsem[p])]
            cps += [
                pltpu.make_async_copy(
                    ctab_h.at[cidx.at[si, i]],
                    crows.at[p, pl.ds(i * IDXW, IDXW)], gsem[p])
                for i in range(NIDX)
            ]
            return cps

        def mk_out(si, p):
            return pltpu.make_async_copy(
                svs[p].at[pl.ds(0, ROWS)],
                out_h.at[pl.ds((wid * BPW + si * S) * R, ROWS)], osem[p])

        for c in mk_gather(0, 0):
            c.start()

        def outer(oi, carry):
            for p in range(2):
                si = oi * 2 + p

                @pl.when(si + 1 < NSUB)
                def _prefetch():
                    for c in mk_gather(si + 1, 1 - p):
                        c.start()

                @pl.when(oi > 0)
                def _drain_out():
                    mk_out(si - 2, p).wait()

                for c in mk_gather(si, p):
                    c.wait()

                @plsc.parallel_loop(0, S)
                def per_b(b):
                    t = [trows[p, b, pl.ds(j * L, L)] for j in range(8)]
                    for r in range(R):
                        row = b * R + r
                        q = [t[j] * crows[p, row, pl.ds(j * L, L)]
                             for j in range(8)]
                        q = [q[2 * j] + q[2 * j + 1] for j in range(4)]
                        acc = (q[0] + q[1]) + (q[2] + q[3])
                        cs = plsc.cumsum(acc)
                        plsc.store_compressed(
                            svs[p].at[pl.ds(row, L)], cs, mask=lastlane)

                mk_out(si, p).start()
            return carry

        lax.fori_loop(0, NSUB // 2, outer, 0)
        mk_out(NSUB - 2, 0).wait()
        mk_out(NSUB - 1, 1).wait()

    return k(tid, cid, ttab, ctab)


RB = B * R // D


def _tc_loss(scores):
    sr = scores.reshape(RB, D)

    def body(s_ref, o_ref):
        row = lax.broadcasted_iota(jnp.int32, (RB, D), 0)
        col = lax.broadcasted_iota(jnp.int32, (RB, D), 1)
        ispos = ((row * D + col) % R) == 0
        s = s_ref[...]
        x = jnp.where(ispos, s, -s)
        ls = jnp.minimum(x, 0.0) - jnp.log1p(jnp.exp(-jnp.abs(x)))
        o_ref[0, 0] = -jnp.sum(ls) / B

    out = pl.pallas_call(
        body,
        out_shape=jax.ShapeDtypeStruct((1, 1), jnp.float32),
        out_specs=pl.BlockSpec(memory_space=pltpu.SMEM),
    )(sr)
    return out[0, 0]


def kernel(target_ids, context_ids, neg_ids, target_table, context_table):
    tid = target_ids.astype(jnp.int32).reshape(NW, BPW)
    cid = jnp.concatenate(
        [context_ids[:, None], neg_ids], axis=1).astype(jnp.int32)
    cid = cid.reshape(NW, NSUB, NIDX, IDXW)
    scores = _sc_scores(tid, cid, target_table, context_table)
    return _tc_loss(scores)
